# Initial kernel scaffold; baseline (speedup 1.0000x reference)
import jax
import jax.numpy as jnp
from jax.experimental import pallas as pl


def kernel(x, edge_index, edge_weights, W_in, b_in, Wc1, bc1, Wc2, bc2, Wc3, bc3, W_out, b_out):
    raise NotImplementedError("write your pallas kernel here")


# speedup vs baseline: 8.3438x; 8.3438x over previous
import dataclasses
import functools

import jax
import jax.numpy as jnp
from jax import lax
from jax.experimental import pallas as pl
from jax.experimental.pallas import tpu as pltpu
from jax.experimental.pallas import tpu_sc as plsc

N = 10000
E = 320000
H = 128
WIN = 168
EWL = 32

NC = 2
NS = 16
NW = NC * NS
LANES = 16

CHUNK = 128
NCHUNK = E // CHUNK
KMAX = -(-NCHUNK // NW)
SUBROWS = 632
NPAD = SUBROWS * NS
ZOFFS = (0, 128, 256, 384, 504)

RB = 1000

_mesh = plsc.VectorSubcoreMesh(core_axis_name="c", subcore_axis_name="s")

_sc_params = pltpu.CompilerParams()
if "needs_layout_passes" in pltpu.CompilerParams.__dataclass_fields__:
    _sc_params = dataclasses.replace(_sc_params, needs_layout_passes=False)


def _build_pattern(ew_hbm, ewv, pat):
    pltpu.sync_copy(ew_hbm, ewv)

    @pl.loop(0, CHUNK)
    def _(r):
        idx = jnp.full((LANES,), lax.rem(r, EWL), jnp.int32)
        pat[r] = plsc.load_gather(ewv, [idx])


_DEG_KW = dict(
    out_type=jax.ShapeDtypeStruct((NC, NPAD, H), jnp.float32),
    mesh=_mesh,
    scratch_types=[
        pltpu.VMEM((EWL,), jnp.float32),
        pltpu.VMEM((CHUNK, H), jnp.float32),
        pltpu.VMEM((CHUNK,), jnp.int32),
        pltpu.VMEM_SHARED((NPAD, H), jnp.float32),
    ],
    compiler_params=_sc_params,
)


def _deg_body(col_hbm, ew_hbm, degp_hbm, ewv, pat, colv, acc):
    ci = lax.axis_index("c")
    si = lax.axis_index("s")
    w = si * NC + ci

    zero16 = jnp.zeros((LANES,), jnp.float32)

    @pl.loop(0, CHUNK)
    def _(i):
        for j in range(H // LANES):
            pat[i, pl.ds(LANES * j, LANES)] = zero16

    for z in ZOFFS:
        pltpu.sync_copy(pat, acc.at[pl.ds(si * SUBROWS + z, CHUNK)])

    pltpu.sync_copy(ew_hbm, ewv)

    @pl.loop(0, CHUNK)
    def _(r):
        idx = jnp.full((LANES,), lax.rem(r, EWL), jnp.int32)
        val = plsc.load_gather(ewv, [idx])
        for j in range(H // LANES):
            pat[r, pl.ds(LANES * j, LANES)] = val

    plsc.subcore_barrier()

    @pl.loop(0, KMAX)
    def _(k):
        c = w + NW * k

        @pl.when(c < NCHUNK)
        def _():
            pltpu.sync_copy(col_hbm.at[pl.ds(c * CHUNK, CHUNK)], colv)
            pltpu.sync_copy(pat, acc.at[colv], add=True)

    plsc.subcore_barrier()
    pltpu.sync_copy(acc.at[pl.ds(si * SUBROWS, SUBROWS)],
                    degp_hbm.at[ci, pl.ds(si * SUBROWS, SUBROWS)])


_CONV_KW = dict(
    out_type=jax.ShapeDtypeStruct((NC, NPAD, H), jnp.float32),
    mesh=_mesh,
    scratch_types=[
        pltpu.VMEM((EWL,), jnp.float32),
        pltpu.VMEM((CHUNK, LANES), jnp.float32),
        pltpu.VMEM((CHUNK,), jnp.int32),
        pltpu.VMEM((CHUNK,), jnp.int32),
        pltpu.VMEM((CHUNK, H), jnp.float32),
        pltpu.VMEM_SHARED((NPAD, H), jnp.float32),
        pltpu.SemaphoreType.DMA,
    ],
    compiler_params=_sc_params,
)


def _conv_body(tab_hbm, row_hbm, col_hbm, ew_hbm, p_hbm,
                 ewv, pat, rowv, colv, rowsv, acc, sem):
    ci = lax.axis_index("c")
    si = lax.axis_index("s")
    w = si * NC + ci

    _build_pattern(ew_hbm, ewv, pat)
    zero16 = jnp.zeros((LANES,), jnp.float32)

    @pl.loop(0, CHUNK)
    def _(i):
        for j in range(H // LANES):
            rowsv[i, pl.ds(LANES * j, LANES)] = zero16

    for z in ZOFFS:
        pltpu.sync_copy(rowsv, acc.at[pl.ds(si * SUBROWS + z, CHUNK)])

    plsc.subcore_barrier()

    @pl.loop(0, KMAX)
    def _(k):
        c = w + NW * k

        @pl.when(c < NCHUNK)
        def _():
            base = c * CHUNK
            pltpu.sync_copy(row_hbm.at[pl.ds(base, CHUNK)], rowv)
            pltpu.sync_copy(col_hbm.at[pl.ds(base, CHUNK)], colv)
            pltpu.async_copy(tab_hbm.at[rowv], rowsv, sem).wait()

            @pl.loop(0, CHUNK)
            def _(r):
                sv = pat[r]
                for j in range(H // LANES):
                    sl = pl.ds(LANES * j, LANES)
                    rowsv[r, sl] = rowsv[r, sl] * sv

            pltpu.sync_copy(rowsv, acc.at[colv], add=True)

    plsc.subcore_barrier()
    pltpu.sync_copy(acc.at[pl.ds(si * SUBROWS, SUBROWS)],
                    p_hbm.at[ci, pl.ds(si * SUBROWS, SUBROWS)])


_deg_kernel = pl.kernel(_deg_body, **_DEG_KW)
_conv_kernel = pl.kernel(_conv_body, **_CONV_KW)


def _dis_block(degp_blk):
    deg = degp_blk[0, :, 0:1] + degp_blk[1, :, 0:1]
    return jnp.where(deg > 0, lax.rsqrt(deg), 0.0)


def _tc_in(x, W_in, b_in):
    def body(x_ref, w_ref, b_ref, h_ref):
        acc = jnp.dot(x_ref[...], w_ref[...],
                      preferred_element_type=jnp.float32)
        h_ref[...] = jnp.maximum(acc + b_ref[...][None, :], 0.0)

    return pl.pallas_call(
        body,
        grid=(N // RB,),
        in_specs=[
            pl.BlockSpec((RB, WIN), lambda i: (i, 0)),
            pl.BlockSpec((WIN, H), lambda i: (0, 0)),
            pl.BlockSpec((H,), lambda i: (0,)),
        ],
        out_specs=pl.BlockSpec((RB, H), lambda i: (i, 0)),
        out_shape=jax.ShapeDtypeStruct((N, H), jnp.float32),
    )(x, W_in, b_in)


def _tc_first_m(h, degp, Wc):
    def body(h_ref, degp_ref, w_ref, m_ref, dis_ref):
        dis = _dis_block(degp_ref[...])
        dis_ref[...] = dis
        m = jnp.dot(h_ref[...], w_ref[...],
                    preferred_element_type=jnp.float32)
        m_ref[...] = m * dis

    return pl.pallas_call(
        body,
        grid=(N // RB,),
        in_specs=[
            pl.BlockSpec((RB, H), lambda i: (i, 0)),
            pl.BlockSpec((NC, RB, H), lambda i: (0, i, 0)),
            pl.BlockSpec((H, H), lambda i: (0, 0)),
        ],
        out_specs=[
            pl.BlockSpec((RB, H), lambda i: (i, 0)),
            pl.BlockSpec((RB, 1), lambda i: (i, 0)),
        ],
        out_shape=[
            jax.ShapeDtypeStruct((N, H), jnp.float32),
            jax.ShapeDtypeStruct((N, 1), jnp.float32),
        ],
    )(h, degp, Wc)


def _tc_mid(P, dis, h, b, Wn):
    def body(p_ref, dis_ref, h_ref, b_ref, w_ref, hn_ref, mn_ref):
        dis = dis_ref[...]
        acc = p_ref[0] + p_ref[1]
        c = acc * dis + b_ref[...][None, :]
        hn = jnp.maximum(c, 0.0) + h_ref[...]
        hn_ref[...] = hn
        mn = jnp.dot(hn, w_ref[...], preferred_element_type=jnp.float32)
        mn_ref[...] = mn * dis

    return pl.pallas_call(
        body,
        grid=(N // RB,),
        in_specs=[
            pl.BlockSpec((NC, RB, H), lambda i: (0, i, 0)),
            pl.BlockSpec((RB, 1), lambda i: (i, 0)),
            pl.BlockSpec((RB, H), lambda i: (i, 0)),
            pl.BlockSpec((H,), lambda i: (0,)),
            pl.BlockSpec((H, H), lambda i: (0, 0)),
        ],
        out_specs=[
            pl.BlockSpec((RB, H), lambda i: (i, 0)),
            pl.BlockSpec((RB, H), lambda i: (i, 0)),
        ],
        out_shape=[
            jax.ShapeDtypeStruct((N, H), jnp.float32),
            jax.ShapeDtypeStruct((N, H), jnp.float32),
        ],
    )(P, dis, h, b, Wn)


def _tc_final(P, dis, h, b, wt, bo):
    def body(p_ref, dis_ref, h_ref, b_ref, wt_ref, bo_ref, o_ref):
        dis = dis_ref[...]
        acc = p_ref[0] + p_ref[1]
        c = acc * dis + b_ref[...][None, :]
        hn = jnp.maximum(c, 0.0) + h_ref[...]
        o_ref[...] = (jnp.sum(hn * wt_ref[...], axis=1, keepdims=True)
                      + bo_ref[...])

    return pl.pallas_call(
        body,
        grid=(N // RB,),
        in_specs=[
            pl.BlockSpec((NC, RB, H), lambda i: (0, i, 0)),
            pl.BlockSpec((RB, 1), lambda i: (i, 0)),
            pl.BlockSpec((RB, H), lambda i: (i, 0)),
            pl.BlockSpec((H,), lambda i: (0,)),
            pl.BlockSpec((1, H), lambda i: (0, 0)),
            pl.BlockSpec((1, 1), lambda i: (0, 0)),
        ],
        out_specs=pl.BlockSpec((RB, 1), lambda i: (i, 0)),
        out_shape=jax.ShapeDtypeStruct((N, 1), jnp.float32),
    )(P, dis, h, b, wt, bo)


def kernel(x, edge_index, edge_weights, W_in, b_in,
           Wc1, bc1, Wc2, bc2, Wc3, bc3, W_out, b_out):
    row = edge_index[0]
    col = edge_index[1]
    ew = jnp.clip(edge_weights, 1e-10, None)

    degp = _deg_kernel(col, ew)
    h = _tc_in(x, W_in, b_in)
    m, dis = _tc_first_m(h, degp, Wc1)
    for (b_k, W_next) in ((bc1, Wc2), (bc2, Wc3)):
        P = _conv_kernel(m, row, col, ew)
        h, m = _tc_mid(P, dis, h, b_k, W_next)
    P = _conv_kernel(m, row, col, ew)
    out = _tc_final(P, dis, h, bc3, W_out.reshape(1, H), b_out.reshape(1, 1))
    return out


# speedup vs baseline: 9.0844x; 1.0888x over previous
import dataclasses
import functools

import jax
import jax.numpy as jnp
from jax import lax
from jax.experimental import pallas as pl
from jax.experimental.pallas import tpu as pltpu
from jax.experimental.pallas import tpu_sc as plsc

N = 10000
E = 320000
H = 128
WIN = 168
EWL = 32

NC = 2
NS = 16
NW = NC * NS
LANES = 16

CHUNK = 128
NCHUNK = E // CHUNK
KMAX = -(-NCHUNK // NW)
SUBROWS = 632
LASTROWS = N - 15 * SUBROWS
ZOFFS = (0, 128, 256, 384, 504)

RB = 1000

_mesh = plsc.VectorSubcoreMesh(core_axis_name="c", subcore_axis_name="s")

_sc_params = pltpu.CompilerParams()
if "needs_layout_passes" in pltpu.CompilerParams.__dataclass_fields__:
    _sc_params = dataclasses.replace(_sc_params, needs_layout_passes=False)


def _build_pattern(ew_hbm, ewv, pat):
    pltpu.sync_copy(ew_hbm, ewv)

    @pl.loop(0, CHUNK)
    def _(r):
        idx = jnp.full((LANES,), lax.rem(r, EWL), jnp.int32)
        pat[r] = plsc.load_gather(ewv, [idx])


_DEG_KW = dict(
    out_type=jax.ShapeDtypeStruct((NC, N, H), jnp.float32),
    mesh=_mesh,
    scratch_types=[
        pltpu.VMEM((EWL,), jnp.float32),
        pltpu.VMEM((CHUNK, H), jnp.float32),
        pltpu.VMEM((CHUNK,), jnp.int32),
        pltpu.VMEM_SHARED((N, H), jnp.float32),
    ],
    compiler_params=_sc_params,
)


def _deg_body(col_hbm, ew_hbm, degp_hbm, ewv, pat, colv, acc):
    ci = lax.axis_index("c")
    si = lax.axis_index("s")
    w = si * NC + ci

    zero16 = jnp.zeros((LANES,), jnp.float32)

    @pl.loop(0, CHUNK)
    def _(i):
        for j in range(H // LANES):
            pat[i, pl.ds(LANES * j, LANES)] = zero16

    zlast = jnp.minimum(si * SUBROWS + ZOFFS[-1], N - CHUNK)
    for z in ZOFFS[:-1]:
        pltpu.sync_copy(pat, acc.at[pl.ds(si * SUBROWS + z, CHUNK)])
    pltpu.sync_copy(pat, acc.at[pl.ds(zlast, CHUNK)])

    pltpu.sync_copy(ew_hbm, ewv)

    @pl.loop(0, CHUNK)
    def _(r):
        idx = jnp.full((LANES,), lax.rem(r, EWL), jnp.int32)
        val = plsc.load_gather(ewv, [idx])
        for j in range(H // LANES):
            pat[r, pl.ds(LANES * j, LANES)] = val

    plsc.subcore_barrier()

    @pl.loop(0, KMAX)
    def _(k):
        c = w + NW * k

        @pl.when(c < NCHUNK)
        def _():
            pltpu.sync_copy(col_hbm.at[pl.ds(c * CHUNK, CHUNK)], colv)
            pltpu.sync_copy(pat, acc.at[colv], add=True)

    plsc.subcore_barrier()

    @pl.when(si < NS - 1)
    def _():
        pltpu.sync_copy(acc.at[pl.ds(si * SUBROWS, SUBROWS)],
                        degp_hbm.at[ci, pl.ds(si * SUBROWS, SUBROWS)])

    @pl.when(si == NS - 1)
    def _():
        pltpu.sync_copy(acc.at[pl.ds((NS - 1) * SUBROWS, LASTROWS)],
                        degp_hbm.at[ci, pl.ds((NS - 1) * SUBROWS, LASTROWS)])


_CONV_KW = dict(
    out_type=jax.ShapeDtypeStruct((NC, N, H), jnp.float32),
    mesh=_mesh,
    scratch_types=[
        pltpu.VMEM((EWL,), jnp.float32),
        pltpu.VMEM((CHUNK, LANES), jnp.float32),
        pltpu.VMEM((4, CHUNK), jnp.int32),
        pltpu.VMEM((4, CHUNK), jnp.int32),
        pltpu.VMEM((2, CHUNK, H), jnp.float32),
        pltpu.VMEM_SHARED((N, H), jnp.float32),
        pltpu.SemaphoreType.DMA((4,)),
        pltpu.SemaphoreType.DMA((2,)),
        pltpu.SemaphoreType.DMA((2,)),
    ],
    compiler_params=_sc_params,
)


def _conv_body(tab_hbm, row_hbm, col_hbm, ew_hbm, p_hbm,
               ewv, pat, rowv4, colv4, rows2, acc, semi, semg, sems):
    ci = lax.axis_index("c")
    si = lax.axis_index("s")
    w = si * NC + ci

    def c_of(k):
        return w + NW * k

    def issue_idx(k, q):
        @pl.when(c_of(k) < NCHUNK)
        def _():
            base = c_of(k) * CHUNK
            pltpu.async_copy(row_hbm.at[pl.ds(base, CHUNK)],
                             rowv4.at[q], semi.at[q])
            pltpu.async_copy(col_hbm.at[pl.ds(base, CHUNK)],
                             colv4.at[q], semi.at[q])

    def wait_idx(k, q):
        @pl.when(c_of(k) < NCHUNK)
        def _():
            base = c_of(k) * CHUNK
            pltpu.make_async_copy(row_hbm.at[pl.ds(base, CHUNK)],
                                  rowv4.at[q], semi.at[q]).wait()
            pltpu.make_async_copy(col_hbm.at[pl.ds(base, CHUNK)],
                                  colv4.at[q], semi.at[q]).wait()

    def issue_gather(k, b, q):
        @pl.when(c_of(k) < NCHUNK)
        def _():
            pltpu.async_copy(tab_hbm.at[rowv4.at[q]], rows2.at[b],
                             semg.at[b])

    def wait_gather(k, b, q):
        @pl.when(c_of(k) < NCHUNK)
        def _():
            pltpu.make_async_copy(tab_hbm.at[rowv4.at[q]], rows2.at[b],
                                  semg.at[b]).wait()

    def issue_scatter(k, b, q):
        @pl.when(c_of(k) < NCHUNK)
        def _():
            pltpu.async_copy(rows2.at[b], acc.at[colv4.at[q]], sems.at[b],
                             add=True)

    def wait_scatter(k, b, q):
        @pl.when((k >= 0) & (c_of(k) < NCHUNK))
        def _():
            pltpu.make_async_copy(rows2.at[b], acc.at[colv4.at[q]],
                                  sems.at[b]).wait()

    zero16 = jnp.zeros((LANES,), jnp.float32)

    @pl.loop(0, CHUNK)
    def _(i):
        for j in range(H // LANES):
            rows2[0, i, pl.ds(LANES * j, LANES)] = zero16

    zlast = jnp.minimum(si * SUBROWS + ZOFFS[-1], N - CHUNK)
    for z in ZOFFS[:-1]:
        pltpu.sync_copy(rows2.at[0], acc.at[pl.ds(si * SUBROWS + z, CHUNK)])
    pltpu.sync_copy(rows2.at[0], acc.at[pl.ds(zlast, CHUNK)])

    plsc.subcore_barrier()

    _build_pattern(ew_hbm, ewv, pat)

    for kk in range(3):
        issue_idx(kk, kk % 4)
    wait_idx(0, 0)
    issue_gather(0, 0, 0)

    @pl.loop(0, KMAX + 1, step=4)
    def _(t):
        for u in range(4):
            k = t + u
            b, q = u % 2, u % 4
            bn, qn = (u + 1) % 2, (u + 1) % 4
            qp = (u + 3) % 4
            wait_idx(k + 1, qn)
            wait_scatter(k - 1, bn, qp)
            issue_gather(k + 1, bn, qn)
            wait_gather(k, b, q)
            issue_idx(k + 3, qp)

            @pl.when(c_of(k) < NCHUNK)
            def _():
                @pl.loop(0, CHUNK, unroll=8)
                def _(r):
                    sv = pat[r]
                    for j in range(H // LANES):
                        sl = pl.ds(LANES * j, LANES)
                        rows2[b, r, sl] = rows2[b, r, sl] * sv

            issue_scatter(k, b, q)

    plsc.subcore_barrier()

    @pl.when(si < NS - 1)
    def _():
        pltpu.sync_copy(acc.at[pl.ds(si * SUBROWS, SUBROWS)],
                        p_hbm.at[ci, pl.ds(si * SUBROWS, SUBROWS)])

    @pl.when(si == NS - 1)
    def _():
        pltpu.sync_copy(acc.at[pl.ds((NS - 1) * SUBROWS, LASTROWS)],
                        p_hbm.at[ci, pl.ds((NS - 1) * SUBROWS, LASTROWS)])


_deg_kernel = pl.kernel(_deg_body, **_DEG_KW)
_conv_kernel = pl.kernel(_conv_body, **_CONV_KW)


def _dis_block(degp_blk):
    deg = degp_blk[0, :, 0:1] + degp_blk[1, :, 0:1]
    return jnp.where(deg > 0, lax.rsqrt(deg), 0.0)


def _tc_in(x, W_in, b_in):
    def body(x_ref, w_ref, b_ref, h_ref):
        acc = jnp.dot(x_ref[...], w_ref[...],
                      preferred_element_type=jnp.float32)
        h_ref[...] = jnp.maximum(acc + b_ref[...][None, :], 0.0)

    return pl.pallas_call(
        body,
        grid=(N // RB,),
        in_specs=[
            pl.BlockSpec((RB, WIN), lambda i: (i, 0)),
            pl.BlockSpec((WIN, H), lambda i: (0, 0)),
            pl.BlockSpec((H,), lambda i: (0,)),
        ],
        out_specs=pl.BlockSpec((RB, H), lambda i: (i, 0)),
        out_shape=jax.ShapeDtypeStruct((N, H), jnp.float32),
    )(x, W_in, b_in)


def _tc_first_m(h, degp, Wc):
    def body(h_ref, degp_ref, w_ref, m_ref, dis_ref):
        dis = _dis_block(degp_ref[...])
        dis_ref[...] = dis
        m = jnp.dot(h_ref[...], w_ref[...],
                    preferred_element_type=jnp.float32)
        m_ref[...] = m * dis

    return pl.pallas_call(
        body,
        grid=(N // RB,),
        in_specs=[
            pl.BlockSpec((RB, H), lambda i: (i, 0)),
            pl.BlockSpec((NC, RB, H), lambda i: (0, i, 0)),
            pl.BlockSpec((H, H), lambda i: (0, 0)),
        ],
        out_specs=[
            pl.BlockSpec((RB, H), lambda i: (i, 0)),
            pl.BlockSpec((RB, 1), lambda i: (i, 0)),
        ],
        out_shape=[
            jax.ShapeDtypeStruct((N, H), jnp.float32),
            jax.ShapeDtypeStruct((N, 1), jnp.float32),
        ],
    )(h, degp, Wc)


def _tc_mid(P, dis, h, b, Wn):
    def body(p_ref, dis_ref, h_ref, b_ref, w_ref, hn_ref, mn_ref):
        dis = dis_ref[...]
        acc = p_ref[0] + p_ref[1]
        c = acc * dis + b_ref[...][None, :]
        hn = jnp.maximum(c, 0.0) + h_ref[...]
        hn_ref[...] = hn
        mn = jnp.dot(hn, w_ref[...], preferred_element_type=jnp.float32)
        mn_ref[...] = mn * dis

    return pl.pallas_call(
        body,
        grid=(N // RB,),
        in_specs=[
            pl.BlockSpec((NC, RB, H), lambda i: (0, i, 0)),
            pl.BlockSpec((RB, 1), lambda i: (i, 0)),
            pl.BlockSpec((RB, H), lambda i: (i, 0)),
            pl.BlockSpec((H,), lambda i: (0,)),
            pl.BlockSpec((H, H), lambda i: (0, 0)),
        ],
        out_specs=[
            pl.BlockSpec((RB, H), lambda i: (i, 0)),
            pl.BlockSpec((RB, H), lambda i: (i, 0)),
        ],
        out_shape=[
            jax.ShapeDtypeStruct((N, H), jnp.float32),
            jax.ShapeDtypeStruct((N, H), jnp.float32),
        ],
    )(P, dis, h, b, Wn)


def _tc_final(P, dis, h, b, wt, bo):
    def body(p_ref, dis_ref, h_ref, b_ref, wt_ref, bo_ref, o_ref):
        dis = dis_ref[...]
        acc = p_ref[0] + p_ref[1]
        c = acc * dis + b_ref[...][None, :]
        hn = jnp.maximum(c, 0.0) + h_ref[...]
        o_ref[...] = (jnp.sum(hn * wt_ref[...], axis=1, keepdims=True)
                      + bo_ref[...])

    return pl.pallas_call(
        body,
        grid=(N // RB,),
        in_specs=[
            pl.BlockSpec((NC, RB, H), lambda i: (0, i, 0)),
            pl.BlockSpec((RB, 1), lambda i: (i, 0)),
            pl.BlockSpec((RB, H), lambda i: (i, 0)),
            pl.BlockSpec((H,), lambda i: (0,)),
            pl.BlockSpec((1, H), lambda i: (0, 0)),
            pl.BlockSpec((1, 1), lambda i: (0, 0)),
        ],
        out_specs=pl.BlockSpec((RB, 1), lambda i: (i, 0)),
        out_shape=jax.ShapeDtypeStruct((N, 1), jnp.float32),
    )(P, dis, h, b, wt, bo)


def kernel(x, edge_index, edge_weights, W_in, b_in,
           Wc1, bc1, Wc2, bc2, Wc3, bc3, W_out, b_out):
    row = edge_index[0]
    col = edge_index[1]
    ew = jnp.clip(edge_weights, 1e-10, None)

    degp = _deg_kernel(col, ew)
    h = _tc_in(x, W_in, b_in)
    m, dis = _tc_first_m(h, degp, Wc1)
    for (b_k, W_next) in ((bc1, Wc2), (bc2, Wc3)):
        P = _conv_kernel(m, row, col, ew)
        h, m = _tc_mid(P, dis, h, b_k, W_next)
    P = _conv_kernel(m, row, col, ew)
    out = _tc_final(P, dis, h, bc3, W_out.reshape(1, H), b_out.reshape(1, 1))
    return out


# speedup vs baseline: 15.2850x; 1.6826x over previous
import dataclasses
import functools

import jax
import jax.numpy as jnp
from jax import lax
from jax.experimental import pallas as pl
from jax.experimental.pallas import tpu as pltpu
from jax.experimental.pallas import tpu_sc as plsc

N = 10000
E = 320000
H = 128
WIN = 168
EWL = 32

NC = 2
NS = 16
NW = NC * NS
LANES = 16

CHUNK = 128
NCHUNK = E // CHUNK
KMAX = -(-NCHUNK // NW)
SUBROWS = 632
LASTROWS = N - 15 * SUBROWS
ZOFFS = (0, 128, 256, 384, 504)

RB = 1000

_mesh = plsc.VectorSubcoreMesh(core_axis_name="c", subcore_axis_name="s")

_sc_params = pltpu.CompilerParams()
if "needs_layout_passes" in pltpu.CompilerParams.__dataclass_fields__:
    _sc_params = dataclasses.replace(_sc_params, needs_layout_passes=False)


def _build_pattern(ew_hbm, ewv, pat):
    pltpu.sync_copy(ew_hbm, ewv)

    @pl.loop(0, CHUNK)
    def _(r):
        idx = jnp.full((LANES,), lax.rem(r, EWL), jnp.int32)
        pat[r] = plsc.load_gather(ewv, [idx])


_DEG_KW = dict(
    out_type=jax.ShapeDtypeStruct((NC, N, H), jnp.float32),
    mesh=_mesh,
    scratch_types=[
        pltpu.VMEM((EWL,), jnp.float32),
        pltpu.VMEM((CHUNK, H), jnp.float32),
        pltpu.VMEM((CHUNK,), jnp.int32),
        pltpu.VMEM_SHARED((N, H), jnp.float32),
    ],
    compiler_params=_sc_params,
)


def _deg_body(col_hbm, ew_hbm, degp_hbm, ewv, pat, colv, acc):
    ci = lax.axis_index("c")
    si = lax.axis_index("s")
    w = si * NC + ci

    zero16 = jnp.zeros((LANES,), jnp.float32)

    @pl.loop(0, CHUNK)
    def _(i):
        for j in range(H // LANES):
            pat[i, pl.ds(LANES * j, LANES)] = zero16

    zlast = jnp.minimum(si * SUBROWS + ZOFFS[-1], N - CHUNK)
    for z in ZOFFS[:-1]:
        pltpu.sync_copy(pat, acc.at[pl.ds(si * SUBROWS + z, CHUNK)])
    pltpu.sync_copy(pat, acc.at[pl.ds(zlast, CHUNK)])

    pltpu.sync_copy(ew_hbm, ewv)

    @pl.loop(0, CHUNK)
    def _(r):
        idx = jnp.full((LANES,), lax.rem(r, EWL), jnp.int32)
        val = plsc.load_gather(ewv, [idx])
        for j in range(H // LANES):
            pat[r, pl.ds(LANES * j, LANES)] = val

    plsc.subcore_barrier()

    @pl.loop(0, KMAX)
    def _(k):
        c = w + NW * k

        @pl.when(c < NCHUNK)
        def _():
            pltpu.sync_copy(col_hbm.at[pl.ds(c * CHUNK, CHUNK)], colv)
            pltpu.sync_copy(pat, acc.at[colv], add=True)

    plsc.subcore_barrier()

    @pl.when(si < NS - 1)
    def _():
        pltpu.sync_copy(acc.at[pl.ds(si * SUBROWS, SUBROWS)],
                        degp_hbm.at[ci, pl.ds(si * SUBROWS, SUBROWS)])

    @pl.when(si == NS - 1)
    def _():
        pltpu.sync_copy(acc.at[pl.ds((NS - 1) * SUBROWS, LASTROWS)],
                        degp_hbm.at[ci, pl.ds((NS - 1) * SUBROWS, LASTROWS)])


_CONV_KW = dict(
    out_type=jax.ShapeDtypeStruct((NC, N, H), jnp.float32),
    mesh=_mesh,
    scratch_types=[
        pltpu.VMEM((EWL,), jnp.float32),
        pltpu.VMEM((CHUNK, LANES), jnp.float32),
        pltpu.VMEM((4, CHUNK), jnp.int32),
        pltpu.VMEM((4, CHUNK), jnp.int32),
        pltpu.VMEM((2, CHUNK, H), jnp.float32),
        pltpu.VMEM_SHARED((N, H), jnp.float32),
        pltpu.SemaphoreType.DMA((4,)),
        pltpu.SemaphoreType.DMA((2,)),
        pltpu.SemaphoreType.DMA((2,)),
    ],
    compiler_params=_sc_params,
)


def _conv_body(tab_hbm, row_hbm, col_hbm, ew_hbm, p_hbm,
               ewv, pat, rowv4, colv4, rows2, acc, semi, semg, sems):
    ci = lax.axis_index("c")
    si = lax.axis_index("s")
    w = si * NC + ci

    def c_of(k):
        return w + NW * k

    def issue_idx(k, q):
        @pl.when(c_of(k) < NCHUNK)
        def _():
            base = c_of(k) * CHUNK
            pltpu.async_copy(row_hbm.at[pl.ds(base, CHUNK)],
                             rowv4.at[q], semi.at[q])
            pltpu.async_copy(col_hbm.at[pl.ds(base, CHUNK)],
                             colv4.at[q], semi.at[q])

    def wait_idx(k, q):
        @pl.when(c_of(k) < NCHUNK)
        def _():
            base = c_of(k) * CHUNK
            pltpu.make_async_copy(row_hbm.at[pl.ds(base, CHUNK)],
                                  rowv4.at[q], semi.at[q]).wait()
            pltpu.make_async_copy(col_hbm.at[pl.ds(base, CHUNK)],
                                  colv4.at[q], semi.at[q]).wait()

    def issue_gather(k, b, q):
        @pl.when(c_of(k) < NCHUNK)
        def _():
            pltpu.async_copy(tab_hbm.at[rowv4.at[q]], rows2.at[b],
                             semg.at[b])

    def wait_gather(k, b, q):
        @pl.when(c_of(k) < NCHUNK)
        def _():
            pltpu.make_async_copy(tab_hbm.at[rowv4.at[q]], rows2.at[b],
                                  semg.at[b]).wait()

    def issue_scatter(k, b, q):
        @pl.when(c_of(k) < NCHUNK)
        def _():
            pltpu.async_copy(rows2.at[b], acc.at[colv4.at[q]], sems.at[b],
                             add=True)

    def wait_scatter(k, b, q):
        @pl.when((k >= 0) & (c_of(k) < NCHUNK))
        def _():
            pltpu.make_async_copy(rows2.at[b], acc.at[colv4.at[q]],
                                  sems.at[b]).wait()

    zero16 = jnp.zeros((LANES,), jnp.float32)

    @pl.loop(0, CHUNK)
    def _(i):
        for j in range(H // LANES):
            rows2[0, i, pl.ds(LANES * j, LANES)] = zero16

    zlast = jnp.minimum(si * SUBROWS + ZOFFS[-1], N - CHUNK)
    for z in ZOFFS[:-1]:
        pltpu.sync_copy(rows2.at[0], acc.at[pl.ds(si * SUBROWS + z, CHUNK)])
    pltpu.sync_copy(rows2.at[0], acc.at[pl.ds(zlast, CHUNK)])

    plsc.subcore_barrier()

    _build_pattern(ew_hbm, ewv, pat)

    for kk in range(3):
        issue_idx(kk, kk % 4)
    wait_idx(0, 0)
    issue_gather(0, 0, 0)

    @pl.loop(0, KMAX + 1, step=4)
    def _(t):
        for u in range(4):
            k = t + u
            b, q = u % 2, u % 4
            bn, qn = (u + 1) % 2, (u + 1) % 4
            qp = (u + 3) % 4
            wait_idx(k + 1, qn)
            wait_scatter(k - 1, bn, qp)
            issue_gather(k + 1, bn, qn)
            wait_gather(k, b, q)
            issue_idx(k + 3, qp)

            @pl.when(c_of(k) < NCHUNK)
            def _():
                @plsc.parallel_loop(0, CHUNK, unroll=8)
                def _(r):
                    sv = pat[r]
                    for j in range(H // LANES):
                        sl = pl.ds(LANES * j, LANES)
                        rows2[b, r, sl] = rows2[b, r, sl] * sv

            issue_scatter(k, b, q)

    plsc.subcore_barrier()

    @pl.when(si < NS - 1)
    def _():
        pltpu.sync_copy(acc.at[pl.ds(si * SUBROWS, SUBROWS)],
                        p_hbm.at[ci, pl.ds(si * SUBROWS, SUBROWS)])

    @pl.when(si == NS - 1)
    def _():
        pltpu.sync_copy(acc.at[pl.ds((NS - 1) * SUBROWS, LASTROWS)],
                        p_hbm.at[ci, pl.ds((NS - 1) * SUBROWS, LASTROWS)])


_deg_kernel = pl.kernel(_deg_body, **_DEG_KW)
_conv_kernel = pl.kernel(_conv_body, **_CONV_KW)


def _dis_block(degp_blk):
    deg = degp_blk[0, :, 0:1] + degp_blk[1, :, 0:1]
    return jnp.where(deg > 0, lax.rsqrt(deg), 0.0)


def _tc_in(x, W_in, b_in):
    def body(x_ref, w_ref, b_ref, h_ref):
        acc = jnp.dot(x_ref[...], w_ref[...],
                      preferred_element_type=jnp.float32)
        h_ref[...] = jnp.maximum(acc + b_ref[...][None, :], 0.0)

    return pl.pallas_call(
        body,
        grid=(N // RB,),
        in_specs=[
            pl.BlockSpec((RB, WIN), lambda i: (i, 0)),
            pl.BlockSpec((WIN, H), lambda i: (0, 0)),
            pl.BlockSpec((H,), lambda i: (0,)),
        ],
        out_specs=pl.BlockSpec((RB, H), lambda i: (i, 0)),
        out_shape=jax.ShapeDtypeStruct((N, H), jnp.float32),
    )(x, W_in, b_in)


def _tc_first_m(h, degp, Wc):
    def body(h_ref, degp_ref, w_ref, m_ref, dis_ref):
        dis = _dis_block(degp_ref[...])
        dis_ref[...] = dis
        m = jnp.dot(h_ref[...], w_ref[...],
                    preferred_element_type=jnp.float32)
        m_ref[...] = m * dis

    return pl.pallas_call(
        body,
        grid=(N // RB,),
        in_specs=[
            pl.BlockSpec((RB, H), lambda i: (i, 0)),
            pl.BlockSpec((NC, RB, H), lambda i: (0, i, 0)),
            pl.BlockSpec((H, H), lambda i: (0, 0)),
        ],
        out_specs=[
            pl.BlockSpec((RB, H), lambda i: (i, 0)),
            pl.BlockSpec((RB, 1), lambda i: (i, 0)),
        ],
        out_shape=[
            jax.ShapeDtypeStruct((N, H), jnp.float32),
            jax.ShapeDtypeStruct((N, 1), jnp.float32),
        ],
    )(h, degp, Wc)


def _tc_mid(P, dis, h, b, Wn):
    def body(p_ref, dis_ref, h_ref, b_ref, w_ref, hn_ref, mn_ref):
        dis = dis_ref[...]
        acc = p_ref[0] + p_ref[1]
        c = acc * dis + b_ref[...][None, :]
        hn = jnp.maximum(c, 0.0) + h_ref[...]
        hn_ref[...] = hn
        mn = jnp.dot(hn, w_ref[...], preferred_element_type=jnp.float32)
        mn_ref[...] = mn * dis

    return pl.pallas_call(
        body,
        grid=(N // RB,),
        in_specs=[
            pl.BlockSpec((NC, RB, H), lambda i: (0, i, 0)),
            pl.BlockSpec((RB, 1), lambda i: (i, 0)),
            pl.BlockSpec((RB, H), lambda i: (i, 0)),
            pl.BlockSpec((H,), lambda i: (0,)),
            pl.BlockSpec((H, H), lambda i: (0, 0)),
        ],
        out_specs=[
            pl.BlockSpec((RB, H), lambda i: (i, 0)),
            pl.BlockSpec((RB, H), lambda i: (i, 0)),
        ],
        out_shape=[
            jax.ShapeDtypeStruct((N, H), jnp.float32),
            jax.ShapeDtypeStruct((N, H), jnp.float32),
        ],
    )(P, dis, h, b, Wn)


def _tc_final(P, dis, h, b, wt, bo):
    def body(p_ref, dis_ref, h_ref, b_ref, wt_ref, bo_ref, o_ref):
        dis = dis_ref[...]
        acc = p_ref[0] + p_ref[1]
        c = acc * dis + b_ref[...][None, :]
        hn = jnp.maximum(c, 0.0) + h_ref[...]
        o_ref[...] = (jnp.sum(hn * wt_ref[...], axis=1, keepdims=True)
                      + bo_ref[...])

    return pl.pallas_call(
        body,
        grid=(N // RB,),
        in_specs=[
            pl.BlockSpec((NC, RB, H), lambda i: (0, i, 0)),
            pl.BlockSpec((RB, 1), lambda i: (i, 0)),
            pl.BlockSpec((RB, H), lambda i: (i, 0)),
            pl.BlockSpec((H,), lambda i: (0,)),
            pl.BlockSpec((1, H), lambda i: (0, 0)),
            pl.BlockSpec((1, 1), lambda i: (0, 0)),
        ],
        out_specs=pl.BlockSpec((RB, 1), lambda i: (i, 0)),
        out_shape=jax.ShapeDtypeStruct((N, 1), jnp.float32),
    )(P, dis, h, b, wt, bo)


def kernel(x, edge_index, edge_weights, W_in, b_in,
           Wc1, bc1, Wc2, bc2, Wc3, bc3, W_out, b_out):
    row = edge_index[0]
    col = edge_index[1]
    ew = jnp.clip(edge_weights, 1e-10, None)

    degp = _deg_kernel(col, ew)
    h = _tc_in(x, W_in, b_in)
    m, dis = _tc_first_m(h, degp, Wc1)
    for (b_k, W_next) in ((bc1, Wc2), (bc2, Wc3)):
        P = _conv_kernel(m, row, col, ew)
        h, m = _tc_mid(P, dis, h, b_k, W_next)
    P = _conv_kernel(m, row, col, ew)
    out = _tc_final(P, dis, h, bc3, W_out.reshape(1, H), b_out.reshape(1, 1))
    return out


# speedup vs baseline: 16.3404x; 1.0690x over previous
import dataclasses
import functools

import jax
import jax.numpy as jnp
from jax import lax
from jax.experimental import pallas as pl
from jax.experimental.pallas import tpu as pltpu
from jax.experimental.pallas import tpu_sc as plsc

N = 10000
E = 320000
H = 128
WIN = 168
EWL = 32

NC = 2
NS = 16
NW = NC * NS
LANES = 16

CHUNK = 128
NCHUNK = E // CHUNK
KMAX = -(-NCHUNK // NW)
SUBROWS = 632
LASTROWS = N - 15 * SUBROWS
ZOFFS = (0, 128, 256, 384, 504)

RB = 1000

_mesh = plsc.VectorSubcoreMesh(core_axis_name="c", subcore_axis_name="s")

_sc_params = pltpu.CompilerParams()
if "needs_layout_passes" in pltpu.CompilerParams.__dataclass_fields__:
    _sc_params = dataclasses.replace(_sc_params, needs_layout_passes=False)


def _build_pattern(ew_hbm, ewv, pat):
    pltpu.sync_copy(ew_hbm, ewv)

    @pl.loop(0, CHUNK)
    def _(r):
        idx = jnp.full((LANES,), lax.rem(r, EWL), jnp.int32)
        pat[r] = plsc.load_gather(ewv, [idx])


_DEG_KW = dict(
    out_type=jax.ShapeDtypeStruct((NC, N, H), jnp.float32),
    mesh=_mesh,
    scratch_types=[
        pltpu.VMEM((EWL,), jnp.float32),
        pltpu.VMEM((CHUNK, H), jnp.float32),
        pltpu.VMEM((8, CHUNK), jnp.int32),
        pltpu.VMEM_SHARED((N, H), jnp.float32),
        pltpu.SemaphoreType.DMA((8,)),
        pltpu.SemaphoreType.DMA((2,)),
    ],
    compiler_params=_sc_params,
)


def _deg_body(col_hbm, ew_hbm, degp_hbm, ewv, pat, colv8, acc, semi, sems):
    ci = lax.axis_index("c")
    si = lax.axis_index("s")
    w = si * NC + ci

    def c_of(k):
        return w + NW * k

    def issue_idx(k, q):
        @pl.when(c_of(k) < NCHUNK)
        def _():
            pltpu.async_copy(col_hbm.at[pl.ds(c_of(k) * CHUNK, CHUNK)],
                             colv8.at[q], semi.at[q])

    def wait_idx(k, q):
        @pl.when(c_of(k) < NCHUNK)
        def _():
            pltpu.make_async_copy(col_hbm.at[pl.ds(c_of(k) * CHUNK, CHUNK)],
                                  colv8.at[q], semi.at[q]).wait()

    def issue_scatter(k, q, b):
        @pl.when(c_of(k) < NCHUNK)
        def _():
            pltpu.async_copy(pat, acc.at[colv8.at[q]], sems.at[b], add=True)

    def wait_scatter(k, q, b):
        @pl.when((k >= 0) & (c_of(k) < NCHUNK))
        def _():
            pltpu.make_async_copy(pat, acc.at[colv8.at[q]],
                                  sems.at[b]).wait()

    zero16 = jnp.zeros((LANES,), jnp.float32)

    @pl.loop(0, CHUNK)
    def _(i):
        for j in range(H // LANES):
            pat[i, pl.ds(LANES * j, LANES)] = zero16

    zlast = jnp.minimum(si * SUBROWS + ZOFFS[-1], N - CHUNK)
    for z in ZOFFS[:-1]:
        pltpu.sync_copy(pat, acc.at[pl.ds(si * SUBROWS + z, CHUNK)])
    pltpu.sync_copy(pat, acc.at[pl.ds(zlast, CHUNK)])

    pltpu.sync_copy(ew_hbm, ewv)

    @pl.loop(0, CHUNK)
    def _(r):
        idx = jnp.full((LANES,), lax.rem(r, EWL), jnp.int32)
        val = plsc.load_gather(ewv, [idx])
        for j in range(H // LANES):
            pat[r, pl.ds(LANES * j, LANES)] = val

    plsc.subcore_barrier()

    for kk in range(6):
        issue_idx(kk, kk % 8)

    @pl.loop(0, 80, step=8)
    def _(t):
        for u in range(8):
            k = t + u
            q, b = u % 8, u % 2
            wait_idx(k, q)
            wait_scatter(k - 2, (u + 6) % 8, b)
            issue_idx(k + 6, (u + 6) % 8)
            issue_scatter(k, q, b)

    wait_scatter(KMAX - 1, (KMAX - 1) % 8, (KMAX - 1) % 2)
    plsc.subcore_barrier()

    @pl.when(si < NS - 1)
    def _():
        pltpu.sync_copy(acc.at[pl.ds(si * SUBROWS, SUBROWS)],
                        degp_hbm.at[ci, pl.ds(si * SUBROWS, SUBROWS)])

    @pl.when(si == NS - 1)
    def _():
        pltpu.sync_copy(acc.at[pl.ds((NS - 1) * SUBROWS, LASTROWS)],
                        degp_hbm.at[ci, pl.ds((NS - 1) * SUBROWS, LASTROWS)])


_CONV_KW = dict(
    out_type=jax.ShapeDtypeStruct((NC, N, H), jnp.float32),
    mesh=_mesh,
    scratch_types=[
        pltpu.VMEM((EWL,), jnp.float32),
        pltpu.VMEM((CHUNK, LANES), jnp.float32),
        pltpu.VMEM((4, CHUNK), jnp.int32),
        pltpu.VMEM((4, CHUNK), jnp.int32),
        pltpu.VMEM((2, CHUNK, H), jnp.float32),
        pltpu.VMEM_SHARED((N, H), jnp.float32),
        pltpu.SemaphoreType.DMA((4,)),
        pltpu.SemaphoreType.DMA((2,)),
        pltpu.SemaphoreType.DMA((2,)),
    ],
    compiler_params=_sc_params,
)


def _conv_body(tab_hbm, row_hbm, col_hbm, ew_hbm, p_hbm,
               ewv, pat, rowv4, colv4, rows2, acc, semi, semg, sems):
    ci = lax.axis_index("c")
    si = lax.axis_index("s")
    w = si * NC + ci

    def c_of(k):
        return w + NW * k

    def issue_idx(k, q):
        @pl.when(c_of(k) < NCHUNK)
        def _():
            base = c_of(k) * CHUNK
            pltpu.async_copy(row_hbm.at[pl.ds(base, CHUNK)],
                             rowv4.at[q], semi.at[q])
            pltpu.async_copy(col_hbm.at[pl.ds(base, CHUNK)],
                             colv4.at[q], semi.at[q])

    def wait_idx(k, q):
        @pl.when(c_of(k) < NCHUNK)
        def _():
            base = c_of(k) * CHUNK
            pltpu.make_async_copy(row_hbm.at[pl.ds(base, CHUNK)],
                                  rowv4.at[q], semi.at[q]).wait()
            pltpu.make_async_copy(col_hbm.at[pl.ds(base, CHUNK)],
                                  colv4.at[q], semi.at[q]).wait()

    def issue_gather(k, b, q):
        @pl.when(c_of(k) < NCHUNK)
        def _():
            pltpu.async_copy(tab_hbm.at[rowv4.at[q]], rows2.at[b],
                             semg.at[b])

    def wait_gather(k, b, q):
        @pl.when(c_of(k) < NCHUNK)
        def _():
            pltpu.make_async_copy(tab_hbm.at[rowv4.at[q]], rows2.at[b],
                                  semg.at[b]).wait()

    def issue_scatter(k, b, q):
        @pl.when(c_of(k) < NCHUNK)
        def _():
            pltpu.async_copy(rows2.at[b], acc.at[colv4.at[q]], sems.at[b],
                             add=True)

    def wait_scatter(k, b, q):
        @pl.when((k >= 0) & (c_of(k) < NCHUNK))
        def _():
            pltpu.make_async_copy(rows2.at[b], acc.at[colv4.at[q]],
                                  sems.at[b]).wait()

    zero16 = jnp.zeros((LANES,), jnp.float32)

    @pl.loop(0, CHUNK)
    def _(i):
        for j in range(H // LANES):
            rows2[0, i, pl.ds(LANES * j, LANES)] = zero16

    zlast = jnp.minimum(si * SUBROWS + ZOFFS[-1], N - CHUNK)
    for z in ZOFFS[:-1]:
        pltpu.sync_copy(rows2.at[0], acc.at[pl.ds(si * SUBROWS + z, CHUNK)])
    pltpu.sync_copy(rows2.at[0], acc.at[pl.ds(zlast, CHUNK)])

    plsc.subcore_barrier()

    _build_pattern(ew_hbm, ewv, pat)

    for kk in range(3):
        issue_idx(kk, kk % 4)
    wait_idx(0, 0)
    issue_gather(0, 0, 0)

    @pl.loop(0, KMAX + 1, step=4)
    def _(t):
        for u in range(4):
            k = t + u
            b, q = u % 2, u % 4
            bn, qn = (u + 1) % 2, (u + 1) % 4
            qp = (u + 3) % 4
            wait_idx(k + 1, qn)
            wait_scatter(k - 1, bn, qp)
            issue_gather(k + 1, bn, qn)
            wait_gather(k, b, q)
            issue_idx(k + 3, qp)

            @pl.when(c_of(k) < NCHUNK)
            def _():
                @plsc.parallel_loop(0, CHUNK, unroll=8)
                def _(r):
                    sv = pat[r]
                    for j in range(H // LANES):
                        sl = pl.ds(LANES * j, LANES)
                        rows2[b, r, sl] = rows2[b, r, sl] * sv

            issue_scatter(k, b, q)

    plsc.subcore_barrier()

    @pl.when(si < NS - 1)
    def _():
        pltpu.sync_copy(acc.at[pl.ds(si * SUBROWS, SUBROWS)],
                        p_hbm.at[ci, pl.ds(si * SUBROWS, SUBROWS)])

    @pl.when(si == NS - 1)
    def _():
        pltpu.sync_copy(acc.at[pl.ds((NS - 1) * SUBROWS, LASTROWS)],
                        p_hbm.at[ci, pl.ds((NS - 1) * SUBROWS, LASTROWS)])


_deg_kernel = pl.kernel(_deg_body, **_DEG_KW)
_conv_kernel = pl.kernel(_conv_body, **_CONV_KW)


def _dis_block(degp_blk):
    deg = degp_blk[0, :, 0:1] + degp_blk[1, :, 0:1]
    return jnp.where(deg > 0, lax.rsqrt(deg), 0.0)


def _tc_in(x, W_in, b_in):
    def body(x_ref, w_ref, b_ref, h_ref):
        acc = jnp.dot(x_ref[...], w_ref[...],
                      preferred_element_type=jnp.float32)
        h_ref[...] = jnp.maximum(acc + b_ref[...][None, :], 0.0)

    return pl.pallas_call(
        body,
        grid=(N // RB,),
        in_specs=[
            pl.BlockSpec((RB, WIN), lambda i: (i, 0)),
            pl.BlockSpec((WIN, H), lambda i: (0, 0)),
            pl.BlockSpec((H,), lambda i: (0,)),
        ],
        out_specs=pl.BlockSpec((RB, H), lambda i: (i, 0)),
        out_shape=jax.ShapeDtypeStruct((N, H), jnp.float32),
    )(x, W_in, b_in)


def _tc_first_m(h, degp, Wc):
    def body(h_ref, degp_ref, w_ref, m_ref, dis_ref):
        dis = _dis_block(degp_ref[...])
        dis_ref[...] = dis
        m = jnp.dot(h_ref[...], w_ref[...],
                    preferred_element_type=jnp.float32)
        m_ref[...] = m * dis

    return pl.pallas_call(
        body,
        grid=(N // RB,),
        in_specs=[
            pl.BlockSpec((RB, H), lambda i: (i, 0)),
            pl.BlockSpec((NC, RB, H), lambda i: (0, i, 0)),
            pl.BlockSpec((H, H), lambda i: (0, 0)),
        ],
        out_specs=[
            pl.BlockSpec((RB, H), lambda i: (i, 0)),
            pl.BlockSpec((RB, 1), lambda i: (i, 0)),
        ],
        out_shape=[
            jax.ShapeDtypeStruct((N, H), jnp.float32),
            jax.ShapeDtypeStruct((N, 1), jnp.float32),
        ],
    )(h, degp, Wc)


def _tc_mid(P, dis, h, b, Wn):
    def body(p_ref, dis_ref, h_ref, b_ref, w_ref, hn_ref, mn_ref):
        dis = dis_ref[...]
        acc = p_ref[0] + p_ref[1]
        c = acc * dis + b_ref[...][None, :]
        hn = jnp.maximum(c, 0.0) + h_ref[...]
        hn_ref[...] = hn
        mn = jnp.dot(hn, w_ref[...], preferred_element_type=jnp.float32)
        mn_ref[...] = mn * dis

    return pl.pallas_call(
        body,
        grid=(N // RB,),
        in_specs=[
            pl.BlockSpec((NC, RB, H), lambda i: (0, i, 0)),
            pl.BlockSpec((RB, 1), lambda i: (i, 0)),
            pl.BlockSpec((RB, H), lambda i: (i, 0)),
            pl.BlockSpec((H,), lambda i: (0,)),
            pl.BlockSpec((H, H), lambda i: (0, 0)),
        ],
        out_specs=[
            pl.BlockSpec((RB, H), lambda i: (i, 0)),
            pl.BlockSpec((RB, H), lambda i: (i, 0)),
        ],
        out_shape=[
            jax.ShapeDtypeStruct((N, H), jnp.float32),
            jax.ShapeDtypeStruct((N, H), jnp.float32),
        ],
    )(P, dis, h, b, Wn)


def _tc_final(P, dis, h, b, wt, bo):
    def body(p_ref, dis_ref, h_ref, b_ref, wt_ref, bo_ref, o_ref):
        dis = dis_ref[...]
        acc = p_ref[0] + p_ref[1]
        c = acc * dis + b_ref[...][None, :]
        hn = jnp.maximum(c, 0.0) + h_ref[...]
        o_ref[...] = (jnp.sum(hn * wt_ref[...], axis=1, keepdims=True)
                      + bo_ref[...])

    return pl.pallas_call(
        body,
        grid=(N // RB,),
        in_specs=[
            pl.BlockSpec((NC, RB, H), lambda i: (0, i, 0)),
            pl.BlockSpec((RB, 1), lambda i: (i, 0)),
            pl.BlockSpec((RB, H), lambda i: (i, 0)),
            pl.BlockSpec((H,), lambda i: (0,)),
            pl.BlockSpec((1, H), lambda i: (0, 0)),
            pl.BlockSpec((1, 1), lambda i: (0, 0)),
        ],
        out_specs=pl.BlockSpec((RB, 1), lambda i: (i, 0)),
        out_shape=jax.ShapeDtypeStruct((N, 1), jnp.float32),
    )(P, dis, h, b, wt, bo)


def kernel(x, edge_index, edge_weights, W_in, b_in,
           Wc1, bc1, Wc2, bc2, Wc3, bc3, W_out, b_out):
    row = edge_index[0]
    col = edge_index[1]
    ew = jnp.clip(edge_weights, 1e-10, None)

    degp = _deg_kernel(col, ew)
    h = _tc_in(x, W_in, b_in)
    m, dis = _tc_first_m(h, degp, Wc1)
    for (b_k, W_next) in ((bc1, Wc2), (bc2, Wc3)):
        P = _conv_kernel(m, row, col, ew)
        h, m = _tc_mid(P, dis, h, b_k, W_next)
    P = _conv_kernel(m, row, col, ew)
    out = _tc_final(P, dis, h, bc3, W_out.reshape(1, H), b_out.reshape(1, 1))
    return out
